# baseline (speedup 1.0000x reference)
import functools
import math

import jax
import jax.numpy as jnp
from jax import lax
from jax.experimental import pallas as pl
from jax.experimental.pallas import tpu as pltpu
from jax.experimental.pallas import tpu_sc as plsc

_TM = 256
_NW = 32
_LANES = 16


def _gelu(x):
    return 0.5 * x * (1.0 + jnp.tanh(math.sqrt(2.0 / math.pi) * (x + 0.044715 * x ** 3)))


def _cumsum_col(v):
    n = v.shape[0]
    k = 1
    zero = jnp.zeros_like(v)
    while k < n:
        v = v + jnp.concatenate([zero[:k], v[:-k]], axis=0)
        k *= 2
    return v


def _gating_body(x_ref, wg_ref, c0_ref, c1_ref, e0_ref, e1_ref, r0_ref, r1_ref,
                 offpad_ref, te_ref, loss_ref, running_ref,
                 *, n_tokens, n_experts, nt_ffn):
    j = pl.program_id(0)
    nsteps = pl.num_programs(0)

    @pl.when(j == 0)
    def _init():
        loss_ref[0, 0] = 0.0
        for e in range(n_experts):
            running_ref[e] = 0

    xt = x_ref[...]
    g = jnp.dot(xt, wg_ref[...], preferred_element_type=jnp.float32)
    loss_ref[0, 0] += jnp.sum(g)

    ids = lax.broadcasted_iota(jnp.int32, g.shape, 1)
    w0 = jnp.max(g, axis=1, keepdims=True)
    e0 = jnp.min(jnp.where(g == w0, ids, n_experts), axis=1, keepdims=True)
    g2 = jnp.where(ids == e0, -jnp.inf, g)
    w1 = jnp.max(g2, axis=1, keepdims=True)
    e1 = jnp.min(jnp.where(g2 == w1, ids, n_experts), axis=1, keepdims=True)
    z = jnp.exp(w1 - w0)
    c0_ref[...] = jnp.broadcast_to(1.0 / (1.0 + z), c0_ref.shape)
    c1_ref[...] = jnp.broadcast_to(z / (1.0 + z), c1_ref.shape)
    e0_ref[...] = e0
    e1_ref[...] = e1

    r0 = jnp.zeros_like(e0)
    r1 = jnp.zeros_like(e0)
    for e in range(n_experts):
        a0 = e0 == e
        a1 = e1 == e
        ce = a0.astype(jnp.int32) + a1.astype(jnp.int32)
        incl = _cumsum_col(ce)
        excl = incl - ce
        base = running_ref[e]
        r0 = jnp.where(a0, base + excl, r0)
        r1 = jnp.where(a1, base + excl + a0.astype(jnp.int32), r1)
        running_ref[e] = base + jnp.sum(ce)
    r0_ref[...] = r0
    r1_ref[...] = r1

    @pl.when(j == nsteps - 1)
    def _fini():
        m = loss_ref[0, 0] / (n_tokens * n_experts)
        loss_ref[0, 0] = m * jnp.log(m + 0.1)
        b = 0
        bt = []
        for e in range(n_experts):
            offpad_ref[0, e] = b * _TM
            b = b + (running_ref[e] + (_TM - 1)) // _TM
            bt.append(b)
        for e in range(n_experts, 16):
            offpad_ref[0, e] = b * _TM

        def _assign(m_, _):
            te = jnp.int32(0)
            for e in range(n_experts - 1):
                te = te + (m_ >= bt[e]).astype(jnp.int32)
            te_ref[0, m_] = te
            return 0

        lax.fori_loop(0, nt_ffn, _assign, 0)


def _gating(flat, Wg, nt_ffn):
    n_tokens, d = flat.shape
    n_experts = Wg.shape[1]
    grid = (n_tokens // _TM,)
    body = functools.partial(_gating_body, n_tokens=n_tokens,
                             n_experts=n_experts, nt_ffn=nt_ffn)
    return pl.pallas_call(
        body,
        grid=grid,
        in_specs=[
            pl.BlockSpec((_TM, d), lambda j: (j, 0)),
            pl.BlockSpec((d, n_experts), lambda j: (0, 0)),
        ],
        out_specs=[
            pl.BlockSpec((_TM, 16), lambda j: (j, 0)),
            pl.BlockSpec((_TM, 16), lambda j: (j, 0)),
            pl.BlockSpec((_TM, 1), lambda j: (j, 0)),
            pl.BlockSpec((_TM, 1), lambda j: (j, 0)),
            pl.BlockSpec((_TM, 1), lambda j: (j, 0)),
            pl.BlockSpec((_TM, 1), lambda j: (j, 0)),
            pl.BlockSpec(memory_space=pltpu.SMEM),
            pl.BlockSpec(memory_space=pltpu.SMEM),
            pl.BlockSpec(memory_space=pltpu.SMEM),
        ],
        out_shape=[
            jax.ShapeDtypeStruct((n_tokens, 16), jnp.float32),
            jax.ShapeDtypeStruct((n_tokens, 16), jnp.float32),
            jax.ShapeDtypeStruct((n_tokens, 1), jnp.int32),
            jax.ShapeDtypeStruct((n_tokens, 1), jnp.int32),
            jax.ShapeDtypeStruct((n_tokens, 1), jnp.int32),
            jax.ShapeDtypeStruct((n_tokens, 1), jnp.int32),
            jax.ShapeDtypeStruct((1, 16), jnp.int32),
            jax.ShapeDtypeStruct((1, nt_ffn), jnp.int32),
            jax.ShapeDtypeStruct((1, 1), jnp.float32),
        ],
        scratch_shapes=[pltpu.SMEM((n_experts,), jnp.int32)],
    )(flat, Wg)


def _dispatch(flat, e0, e1, r0, r1, offpad, n_pad):
    n_tokens, d = flat.shape
    tpw = n_tokens // _NW
    c = min(64, tpw)
    nch = tpw // c
    mesh = plsc.VectorSubcoreMesh(core_axis_name="c", subcore_axis_name="s", num_cores=2, num_subcores=16)
    nc = mesh.num_cores

    @functools.partial(
        pl.kernel, mesh=mesh,
        out_type=[
            jax.ShapeDtypeStruct((n_pad, d), jnp.float32),
            jax.ShapeDtypeStruct((n_tokens,), jnp.int32),
            jax.ShapeDtypeStruct((n_tokens,), jnp.int32),
        ],
        scratch_types=[
            pltpu.VMEM((c, d), jnp.float32),
            pltpu.VMEM((c,), jnp.int32),
            pltpu.VMEM((c,), jnp.int32),
            pltpu.VMEM((c,), jnp.int32),
            pltpu.VMEM((c,), jnp.int32),
            pltpu.VMEM((c,), jnp.int32),
            pltpu.VMEM((c,), jnp.int32),
            pltpu.VMEM((16,), jnp.int32),
            pltpu.SemaphoreType.DMA,
        ],
    )
    def k(x_hbm, e0_hbm, e1_hbm, r0_hbm, r1_hbm, off_hbm,
          xs_hbm, p0_hbm, p1_hbm,
          rows_v, e0_v, e1_v, r0_v, r1_v, p0_v, p1_v, off_v, sem):
        wid = lax.axis_index("s") * nc + lax.axis_index("c")
        base = wid * tpw
        pltpu.sync_copy(off_hbm, off_v)

        def chunk(ci, _):
            tb = base + ci * c
            pltpu.sync_copy(e0_hbm.at[pl.ds(tb, c)], e0_v)
            pltpu.sync_copy(e1_hbm.at[pl.ds(tb, c)], e1_v)
            pltpu.sync_copy(r0_hbm.at[pl.ds(tb, c)], r0_v)
            pltpu.sync_copy(r1_hbm.at[pl.ds(tb, c)], r1_v)
            off_vec = off_v[...]
            offs = [off_vec[e] for e in range(4)]

            def lookup(ev):
                res = jnp.full((_LANES,), offs[0], jnp.int32)
                for e in range(1, 4):
                    res = jnp.where(ev == e, offs[e], res)
                return res

            for kk in range(c // _LANES):
                sl = pl.ds(kk * _LANES, _LANES)
                p0_v[sl] = lookup(e0_v[sl]) + r0_v[sl]
                p1_v[sl] = lookup(e1_v[sl]) + r1_v[sl]
            pltpu.sync_copy(x_hbm.at[pl.ds(tb, c)], rows_v)
            pltpu.async_copy(rows_v, xs_hbm.at[p0_v], sem).wait()
            pltpu.async_copy(rows_v, xs_hbm.at[p1_v], sem).wait()
            pltpu.sync_copy(p0_v, p0_hbm.at[pl.ds(tb, c)])
            pltpu.sync_copy(p1_v, p1_hbm.at[pl.ds(tb, c)])
            return 0

        lax.fori_loop(0, nch, chunk, 0)

    return k(flat, e0, e1, r0, r1, offpad)


def _ffn_body(te_ref, xs_ref, w1_ref, b1_ref, w2_ref, b2_ref, ys_ref):
    h = _gelu(jnp.dot(xs_ref[...], w1_ref[0], preferred_element_type=jnp.float32)
              + b1_ref[0])
    ys_ref[...] = (jnp.dot(h, w2_ref[0], preferred_element_type=jnp.float32)
                   + b2_ref[0])


def _ffn(xs, te, W1, b1, W2, b2):
    n_pad, d = xs.shape
    n_experts, _, d_ff = W1.shape
    nt = n_pad // _TM
    grid_spec = pltpu.PrefetchScalarGridSpec(
        num_scalar_prefetch=1,
        grid=(nt,),
        in_specs=[
            pl.BlockSpec((_TM, d), lambda j, te: (j, 0)),
            pl.BlockSpec((1, d, d_ff), lambda j, te: (te[j], 0, 0)),
            pl.BlockSpec((1, 1, d_ff), lambda j, te: (te[j], 0, 0)),
            pl.BlockSpec((1, d_ff, d), lambda j, te: (te[j], 0, 0)),
            pl.BlockSpec((1, 1, d), lambda j, te: (te[j], 0, 0)),
        ],
        out_specs=pl.BlockSpec((_TM, d), lambda j, te: (j, 0)),
    )
    return pl.pallas_call(
        _ffn_body,
        grid_spec=grid_spec,
        out_shape=jax.ShapeDtypeStruct((n_pad, d), jnp.float32),
    )(te, xs, W1, b1.reshape(n_experts, 1, d_ff), W2, b2.reshape(n_experts, 1, d))


def _combine(ys, pos0, pos1, c0, c1, n_tokens, d):
    tpw = n_tokens // _NW
    c = min(32, tpw)
    nch = tpw // c
    mesh = plsc.VectorSubcoreMesh(core_axis_name="c", subcore_axis_name="s", num_cores=2, num_subcores=16)
    nc = mesh.num_cores

    @functools.partial(
        pl.kernel, mesh=mesh,
        out_type=jax.ShapeDtypeStruct((n_tokens, d), jnp.float32),
        scratch_types=[
            pltpu.VMEM((c, d), jnp.float32),
            pltpu.VMEM((c, d), jnp.float32),
            pltpu.VMEM((c, d), jnp.float32),
            pltpu.VMEM((c,), jnp.int32),
            pltpu.VMEM((c,), jnp.int32),
            pltpu.VMEM((c, 16), jnp.float32),
            pltpu.VMEM((c, 16), jnp.float32),
            pltpu.SemaphoreType.DMA,
        ],
    )
    def k(ys_hbm, p0_hbm, p1_hbm, c0_hbm, c1_hbm, out_hbm,
          g0_v, g1_v, ob_v, p0_v, p1_v, c0_v, c1_v, sem):
        wid = lax.axis_index("s") * nc + lax.axis_index("c")
        base = wid * tpw

        def chunk(ci, _):
            tb = base + ci * c
            pltpu.sync_copy(p0_hbm.at[pl.ds(tb, c)], p0_v)
            pltpu.sync_copy(p1_hbm.at[pl.ds(tb, c)], p1_v)
            pltpu.sync_copy(c0_hbm.at[pl.ds(tb, c)], c0_v)
            pltpu.sync_copy(c1_hbm.at[pl.ds(tb, c)], c1_v)
            pltpu.async_copy(ys_hbm.at[p0_v], g0_v, sem).wait()
            pltpu.async_copy(ys_hbm.at[p1_v], g1_v, sem).wait()

            def tok(t, _):
                c0b = c0_v[t]
                c1b = c1_v[t]
                for v in range(d // _LANES):
                    sl = pl.ds(v * _LANES, _LANES)
                    ob_v[t, sl] = c0b * g0_v[t, sl] + c1b * g1_v[t, sl]
                return 0

            lax.fori_loop(0, c, tok, 0)
            pltpu.sync_copy(ob_v, out_hbm.at[pl.ds(tb, c)])
            return 0

        lax.fori_loop(0, nch, chunk, 0)

    return k(ys, pos0, pos1, c0, c1)


def kernel(x, Wg, W1, b1, W2, b2):
    orig_shape = x.shape
    d = x.shape[-1]
    flat = x.reshape(-1, d)
    n_tokens = flat.shape[0]
    n_experts = W1.shape[0]
    top_k = 2
    n_pad = n_tokens * top_k + n_experts * _TM
    nt_ffn = n_pad // _TM

    c0, c1, e0, e1, r0, r1, offpad, te, loss = _gating(flat, Wg, nt_ffn)
    xs, pos0, pos1 = _dispatch(
        flat, e0.reshape(-1), e1.reshape(-1), r0.reshape(-1), r1.reshape(-1),
        offpad.reshape(-1), n_pad)
    ys = _ffn(xs, te.reshape(-1), W1, b1, W2, b2)
    out = _combine(ys, pos0, pos1, c0, c1, n_tokens, d)
    return out.reshape(orig_shape), loss.reshape(())
